# baseline (speedup 1.0000x reference)
import functools

import jax
import jax.numpy as jnp
from jax.experimental import pallas as pl

B = 64
A = 5
C = 20
H = 19
W = 19
S = 20
HW = H * W
NB = A * HW
IOU_THRESHOLD = 0.6
LAMBDA_OBJ = 5.0
LAMBDA_NOOBJ = 1.0
LAMBDA_COORD = 1.0
IMG = 16


def _image_loss(p, g, anc, lanc, gcx_row, gcy_row, jcol, ci):
    f32 = jnp.float32
    dxg = g[:, 0:1]
    dyg = g[:, 1:2]
    gxs = g[:, 2:3]
    gys = g[:, 3:4]
    wg = g[:, 4:5]
    hg = g[:, 5:6]
    clsg = g[:, 6:7]

    cxg = dxg + gxs / W
    cyg = dyg + gys / H
    gx1 = cxg - wg / 2.0
    gy1 = cyg - hg / 2.0
    gx2 = cxg + wg / 2.0
    gy2 = cyg + hg / 2.0
    area_g = (gx2 - gx1) * (gy2 - gy1)

    ious = []
    overs = []
    m_s = None
    for a in range(A):
        aw = anc[0, a]
        ah = anc[1, a]
        ax1 = gcx_row - aw / 2.0
        ay1 = gcy_row - ah / 2.0
        ax2 = gcx_row + aw / 2.0
        ay2 = gcy_row + ah / 2.0
        area_a = (ax2 - ax1) * (ay2 - ay1)
        iw_a = jnp.clip(jnp.minimum(gx2, ax2) - jnp.maximum(gx1, ax1),
                        0.0, None)
        ih_a = jnp.clip(jnp.minimum(gy2, ay2) - jnp.maximum(gy1, ay1),
                        0.0, None)
        inter_a = iw_a * ih_a
        iou_a = inter_a / (area_g + area_a - inter_a)
        ious.append(iou_a)
        overs.append(jnp.max(iou_a, axis=0, keepdims=True) > IOU_THRESHOLD)
        rm = jnp.max(iou_a, axis=1, keepdims=True)
        m_s = rm if m_s is None else jnp.maximum(m_s, rm)

    bp = None
    for a in range(A):
        cand = jnp.min(jnp.where(ious[a] == m_s, jcol, NB),
                       axis=1, keepdims=True) + a * HW
        bp = cand if bp is None else jnp.minimum(bp, cand)
    bp_f = bp.astype(f32)

    ones_col = jnp.full((S, 1), 1.0, dtype=f32)
    tab = jnp.concatenate(
        [dxg, dyg, jnp.log(wg), jnp.log(hg), clsg, m_s, ones_col, bp_f],
        axis=1)
    tabT = tab.T
    bp_lane = tabT[7:8, :]
    s_sub = jax.lax.broadcasted_iota(jnp.int32, (S, 1), 0)
    s_lane = jax.lax.broadcasted_iota(jnp.int32, (1, S), 1)
    dup = jnp.max(jnp.where((bp_f == bp_lane) & (s_sub > s_lane),
                            1.0, 0.0), axis=0, keepdims=True)
    lhs = tabT * (1.0 - dup)

    contrib = jnp.zeros((1, HW), dtype=f32)
    for a in range(A):
        base = a * (5 + C)
        hit_a = jnp.where(bp == (jcol + a * HW), 1.0, 0.0)
        tm = jnp.dot(lhs, hit_a, preferred_element_type=f32)

        b_a = tm[6:7, :]
        neg_a = jnp.where(overs[a], 0.0, 1.0 - b_a)

        s0 = jax.nn.sigmoid(p[base + 0:base + 1, :])
        s1 = jax.nn.sigmoid(p[base + 1:base + 2, :])
        p2 = p[base + 2:base + 3, :]
        p3 = p[base + 3:base + 4, :]
        s4 = jax.nn.sigmoid(p[base + 4:base + 5, :])
        pc = p[base + 5:base + 25, :]
        mx = jnp.max(pc, axis=0, keepdims=True)
        e = jnp.exp(pc - mx)
        inv = 1.0 / jnp.sum(e, axis=0, keepdims=True)
        tci = tm[4:5, :].astype(jnp.int32)
        e_sel = jnp.sum(jnp.where(ci == tci, e, 0.0), axis=0, keepdims=True)
        e_sq = jnp.sum(e * e, axis=0, keepdims=True)

        t2 = tm[2:3, :] - lanc[0, a]
        t3 = tm[3:4, :] - lanc[1, a]
        coord_t = ((s0 - tm[0:1, :]) ** 2 + (s1 - tm[1:2, :]) ** 2
                   + (p2 - t2) ** 2 + (p3 - t3) ** 2)
        obj_t = (s4 - tm[5:6, :]) ** 2
        cls_t = e_sq * inv * inv - 2.0 * e_sel * inv + 1.0
        contrib = contrib + (LAMBDA_NOOBJ * (neg_a * (s4 * s4))
                             + b_a * (LAMBDA_COORD * coord_t
                                      + LAMBDA_OBJ * obj_t + cls_t))
    return jnp.sum(contrib)


def _loss_kernel(pred_ref, gt_ref, anc_ref, out_ref):
    f32 = jnp.float32
    anc = anc_ref[...].T
    lanc = jnp.log(anc)
    jcol = jax.lax.broadcasted_iota(jnp.int32, (1, HW), 1)
    gcx_row = ((jcol % W).astype(f32) + 0.5) / W
    gcy_row = ((jcol // W).astype(f32) + 0.5) / H
    ci = jax.lax.broadcasted_iota(jnp.int32, (C, HW), 0)
    losses = [
        _image_loss(pred_ref[img], gt_ref[img], anc, lanc,
                    gcx_row, gcy_row, jcol, ci)
        for img in range(IMG)
    ]
    out_ref[...] = jnp.stack(losses).reshape(IMG, 1, 1)


@functools.partial(jax.jit, static_argnames=("interpret",))
def kernel(pred, gt_flat, spans, anchors, interpret=False):
    del spans
    pred3 = pred.reshape(B, A * (5 + C), HW)
    gt3 = gt_flat.reshape(B, S, 7)
    partial = pl.pallas_call(
        _loss_kernel,
        grid=(B // IMG,),
        in_specs=[
            pl.BlockSpec((IMG, A * (5 + C), HW), lambda i: (i, 0, 0)),
            pl.BlockSpec((IMG, S, 7), lambda i: (i, 0, 0)),
            pl.BlockSpec((A, 2), lambda i: (0, 0)),
        ],
        out_specs=pl.BlockSpec((IMG, 1, 1), lambda i: (i, 0, 0)),
        out_shape=jax.ShapeDtypeStruct((B, 1, 1), jnp.float32),
        interpret=interpret,
    )(pred3, gt3, anchors)
    return jnp.sum(partial)


# speedup vs baseline: 1.2826x; 1.2826x over previous
import functools

import jax
import jax.numpy as jnp
from jax.experimental import pallas as pl

B = 64
A = 5
C = 20
H = 19
W = 19
S = 20
HW = H * W
NB = A * HW
IOU_THRESHOLD = 0.6
LAMBDA_OBJ = 5.0
LAMBDA_NOOBJ = 1.0
LAMBDA_COORD = 1.0
IMG = 4


def _image_loss(p, g, anc, lanc, gcx_row, gcy_row, jcol, ci):
    f32 = jnp.float32
    dxg = g[:, 0:1]
    dyg = g[:, 1:2]
    gxs = g[:, 2:3]
    gys = g[:, 3:4]
    wg = g[:, 4:5]
    hg = g[:, 5:6]
    clsg = g[:, 6:7]

    cxg = dxg + gxs / W
    cyg = dyg + gys / H
    gx1 = cxg - wg / 2.0
    gy1 = cyg - hg / 2.0
    gx2 = cxg + wg / 2.0
    gy2 = cyg + hg / 2.0
    area_g = (gx2 - gx1) * (gy2 - gy1)

    ious = []
    overs = []
    m_s = None
    for a in range(A):
        aw = anc[0, a]
        ah = anc[1, a]
        ax1 = gcx_row - aw / 2.0
        ay1 = gcy_row - ah / 2.0
        ax2 = gcx_row + aw / 2.0
        ay2 = gcy_row + ah / 2.0
        area_a = (ax2 - ax1) * (ay2 - ay1)
        iw_a = jnp.clip(jnp.minimum(gx2, ax2) - jnp.maximum(gx1, ax1),
                        0.0, None)
        ih_a = jnp.clip(jnp.minimum(gy2, ay2) - jnp.maximum(gy1, ay1),
                        0.0, None)
        inter_a = iw_a * ih_a
        iou_a = inter_a / (area_g + area_a - inter_a)
        ious.append(iou_a)
        overs.append(jnp.max(iou_a, axis=0, keepdims=True) > IOU_THRESHOLD)
        rm = jnp.max(iou_a, axis=1, keepdims=True)
        m_s = rm if m_s is None else jnp.maximum(m_s, rm)

    bp = None
    for a in range(A):
        cand = jnp.min(jnp.where(ious[a] == m_s, jcol, NB),
                       axis=1, keepdims=True) + a * HW
        bp = cand if bp is None else jnp.minimum(bp, cand)
    bp_f = bp.astype(f32)

    ones_col = jnp.full((S, 1), 1.0, dtype=f32)
    tab = jnp.concatenate(
        [dxg, dyg, jnp.log(wg), jnp.log(hg), clsg, m_s, ones_col, bp_f],
        axis=1)
    tabT = tab.T
    bp_lane = tabT[7:8, :]
    s_sub = jax.lax.broadcasted_iota(jnp.int32, (S, 1), 0)
    s_lane = jax.lax.broadcasted_iota(jnp.int32, (1, S), 1)
    dup = jnp.max(jnp.where((bp_f == bp_lane) & (s_sub > s_lane),
                            1.0, 0.0), axis=0, keepdims=True)
    lhs = tabT * (1.0 - dup)

    contrib = jnp.zeros((1, HW), dtype=f32)
    for a in range(A):
        base = a * (5 + C)
        hit_a = jnp.where(bp == (jcol + a * HW), 1.0, 0.0)
        tm = jnp.dot(lhs, hit_a, preferred_element_type=f32)

        b_a = tm[6:7, :]
        neg_a = jnp.where(overs[a], 0.0, 1.0 - b_a)

        s0 = jax.nn.sigmoid(p[base + 0:base + 1, :])
        s1 = jax.nn.sigmoid(p[base + 1:base + 2, :])
        p2 = p[base + 2:base + 3, :]
        p3 = p[base + 3:base + 4, :]
        s4 = jax.nn.sigmoid(p[base + 4:base + 5, :])
        pc = p[base + 5:base + 25, :]
        mx = jnp.max(pc, axis=0, keepdims=True)
        e = jnp.exp(pc - mx)
        inv = 1.0 / jnp.sum(e, axis=0, keepdims=True)
        tci = tm[4:5, :].astype(jnp.int32)
        e_sel = jnp.sum(jnp.where(ci == tci, e, 0.0), axis=0, keepdims=True)
        e_sq = jnp.sum(e * e, axis=0, keepdims=True)

        t2 = tm[2:3, :] - lanc[0, a]
        t3 = tm[3:4, :] - lanc[1, a]
        coord_t = ((s0 - tm[0:1, :]) ** 2 + (s1 - tm[1:2, :]) ** 2
                   + (p2 - t2) ** 2 + (p3 - t3) ** 2)
        obj_t = (s4 - tm[5:6, :]) ** 2
        cls_t = e_sq * inv * inv - 2.0 * e_sel * inv + 1.0
        contrib = contrib + (LAMBDA_NOOBJ * (neg_a * (s4 * s4))
                             + b_a * (LAMBDA_COORD * coord_t
                                      + LAMBDA_OBJ * obj_t + cls_t))
    return jnp.sum(contrib)


def _loss_kernel(pred_ref, gt_ref, anc_ref, out_ref):
    f32 = jnp.float32
    anc = anc_ref[...].T
    lanc = jnp.log(anc)
    jcol = jax.lax.broadcasted_iota(jnp.int32, (1, HW), 1)
    gcx_row = ((jcol % W).astype(f32) + 0.5) / W
    gcy_row = ((jcol // W).astype(f32) + 0.5) / H
    ci = jax.lax.broadcasted_iota(jnp.int32, (C, HW), 0)
    losses = [
        _image_loss(pred_ref[img], gt_ref[img], anc, lanc,
                    gcx_row, gcy_row, jcol, ci)
        for img in range(IMG)
    ]
    out_ref[...] = jnp.stack(losses).reshape(IMG, 1, 1)


@functools.partial(jax.jit, static_argnames=("interpret",))
def kernel(pred, gt_flat, spans, anchors, interpret=False):
    del spans
    pred3 = pred.reshape(B, A * (5 + C), HW)
    gt3 = gt_flat.reshape(B, S, 7)
    partial = pl.pallas_call(
        _loss_kernel,
        grid=(B // IMG,),
        in_specs=[
            pl.BlockSpec((IMG, A * (5 + C), HW), lambda i: (i, 0, 0)),
            pl.BlockSpec((IMG, S, 7), lambda i: (i, 0, 0)),
            pl.BlockSpec((A, 2), lambda i: (0, 0)),
        ],
        out_specs=pl.BlockSpec((IMG, 1, 1), lambda i: (i, 0, 0)),
        out_shape=jax.ShapeDtypeStruct((B, 1, 1), jnp.float32),
        interpret=interpret,
    )(pred3, gt3, anchors)
    return jnp.sum(partial)


# speedup vs baseline: 1.3461x; 1.0494x over previous
import functools

import jax
import jax.numpy as jnp
from jax.experimental import pallas as pl

B = 64
A = 5
C = 20
H = 19
W = 19
S = 20
HW = H * W
NB = A * HW
IOU_THRESHOLD = 0.6
LAMBDA_OBJ = 5.0
LAMBDA_NOOBJ = 1.0
LAMBDA_COORD = 1.0
IMG = 4


def _image_loss(p, g, anc, lanc, gcx_row, gcy_row, jcol, ci):
    f32 = jnp.float32
    dxg = g[:, 0:1]
    dyg = g[:, 1:2]
    gxs = g[:, 2:3]
    gys = g[:, 3:4]
    wg = g[:, 4:5]
    hg = g[:, 5:6]
    clsg = g[:, 6:7]

    cxg = dxg + gxs / W
    cyg = dyg + gys / H
    gx1 = cxg - wg / 2.0
    gy1 = cyg - hg / 2.0
    gx2 = cxg + wg / 2.0
    gy2 = cyg + hg / 2.0
    area_g = (gx2 - gx1) * (gy2 - gy1)

    ious = []
    overs = []
    m_s = None
    ones_s = jnp.where(
        jax.lax.broadcasted_iota(jnp.int32, (8, S), 0) == 0, 1.0, 0.0)
    for a in range(A):
        aw = anc[0, a]
        ah = anc[1, a]
        ax1 = gcx_row - aw / 2.0
        ay1 = gcy_row - ah / 2.0
        ax2 = gcx_row + aw / 2.0
        ay2 = gcy_row + ah / 2.0
        area_a = (ax2 - ax1) * (ay2 - ay1)
        iw_a = jnp.clip(jnp.minimum(gx2, ax2) - jnp.maximum(gx1, ax1),
                        0.0, None)
        ih_a = jnp.clip(jnp.minimum(gy2, ay2) - jnp.maximum(gy1, ay1),
                        0.0, None)
        inter_a = iw_a * ih_a
        iou_a = inter_a / (area_g + area_a - inter_a)
        ious.append(iou_a)
        ind_a = jnp.where(iou_a > IOU_THRESHOLD, 1.0, 0.0)
        overs.append(jnp.dot(ones_s, ind_a,
                             preferred_element_type=f32)[0:1, :] > 0.0)
        rm = jnp.max(iou_a, axis=1, keepdims=True)
        m_s = rm if m_s is None else jnp.maximum(m_s, rm)

    enc = None
    for a in range(A):
        enc_a = jnp.where(ious[a] == m_s, jcol + a * HW, NB)
        enc = enc_a if enc is None else jnp.minimum(enc, enc_a)
    bp = jnp.min(enc, axis=1, keepdims=True)
    bp_f = bp.astype(f32)

    ones_col = jnp.full((S, 1), 1.0, dtype=f32)
    tab = jnp.concatenate(
        [dxg, dyg, jnp.log(wg), jnp.log(hg), clsg, m_s, ones_col, bp_f],
        axis=1)
    tabT = tab.T
    bp_lane = tabT[7:8, :]
    s_sub = jax.lax.broadcasted_iota(jnp.int32, (S, 1), 0)
    s_lane = jax.lax.broadcasted_iota(jnp.int32, (1, S), 1)
    dup = jnp.max(jnp.where((bp_f == bp_lane) & (s_sub > s_lane),
                            1.0, 0.0), axis=0, keepdims=True)
    lhs = tabT * (1.0 - dup)

    contrib = jnp.zeros((1, HW), dtype=f32)
    for a in range(A):
        base = a * (5 + C)
        hit_a = jnp.where(bp == (jcol + a * HW), 1.0, 0.0)
        tm = jnp.dot(lhs, hit_a, preferred_element_type=f32)

        b_a = tm[6:7, :]
        neg_a = jnp.where(overs[a], 0.0, 1.0 - b_a)

        s0 = jax.nn.sigmoid(p[base + 0:base + 1, :])
        s1 = jax.nn.sigmoid(p[base + 1:base + 2, :])
        p2 = p[base + 2:base + 3, :]
        p3 = p[base + 3:base + 4, :]
        s4 = jax.nn.sigmoid(p[base + 4:base + 5, :])
        pc = p[base + 5:base + 25, :]
        mx = jnp.max(pc, axis=0, keepdims=True)
        e = jnp.exp(pc - mx)
        tci = tm[4:5, :].astype(jnp.int32)
        e_sel = jnp.sum(jnp.where(ci == tci, e, 0.0), axis=0, keepdims=True)
        e_sq = jnp.sum(e * e, axis=0, keepdims=True)
        inv = 1.0 / jnp.sum(e, axis=0, keepdims=True)

        t2 = tm[2:3, :] - lanc[0, a]
        t3 = tm[3:4, :] - lanc[1, a]
        coord_t = ((s0 - tm[0:1, :]) ** 2 + (s1 - tm[1:2, :]) ** 2
                   + (p2 - t2) ** 2 + (p3 - t3) ** 2)
        obj_t = (s4 - tm[5:6, :]) ** 2
        cls_t = e_sq * inv * inv - 2.0 * e_sel * inv + 1.0
        contrib = contrib + (LAMBDA_NOOBJ * (neg_a * (s4 * s4))
                             + b_a * (LAMBDA_COORD * coord_t
                                      + LAMBDA_OBJ * obj_t + cls_t))
    return jnp.sum(contrib)


def _loss_kernel(pred_ref, gt_ref, anc_ref, out_ref):
    f32 = jnp.float32
    anc = anc_ref[...].T
    lanc = jnp.log(anc)
    jcol = jax.lax.broadcasted_iota(jnp.int32, (1, HW), 1)
    gcx_row = ((jcol % W).astype(f32) + 0.5) / W
    gcy_row = ((jcol // W).astype(f32) + 0.5) / H
    ci = jax.lax.broadcasted_iota(jnp.int32, (C, HW), 0)
    losses = [
        _image_loss(pred_ref[img], gt_ref[img], anc, lanc,
                    gcx_row, gcy_row, jcol, ci)
        for img in range(IMG)
    ]
    total = sum(losses).reshape(1, 1)

    @pl.when(pl.program_id(0) == 0)
    def _():
        out_ref[...] = jnp.zeros((1, 1), jnp.float32)

    out_ref[...] += total


@functools.partial(jax.jit, static_argnames=("interpret",))
def kernel(pred, gt_flat, spans, anchors, interpret=False):
    del spans
    pred3 = pred.reshape(B, A * (5 + C), HW)
    gt3 = gt_flat.reshape(B, S, 7)
    partial = pl.pallas_call(
        _loss_kernel,
        grid=(B // IMG,),
        in_specs=[
            pl.BlockSpec((IMG, A * (5 + C), HW), lambda i: (i, 0, 0)),
            pl.BlockSpec((IMG, S, 7), lambda i: (i, 0, 0)),
            pl.BlockSpec((A, 2), lambda i: (0, 0)),
        ],
        out_specs=pl.BlockSpec((1, 1), lambda i: (0, 0)),
        out_shape=jax.ShapeDtypeStruct((1, 1), jnp.float32),
        interpret=interpret,
    )(pred3, gt3, anchors)
    return partial.reshape(())
